# baseline (device time: 612008 ns/iter reference)
import jax
import jax.numpy as jnp
from jax import lax
from jax.experimental import pallas as pl
from jax.experimental.pallas import tpu as pltpu

TN = 256
S_HALF = 1024


def kernel(O, Wo):
    B, S, H, D = O.shape
    K = H * D
    N = Wo.shape[1]
    NT = (N // 2) // TN

    def body(o_ref, w_ref, out_ref, ptile, recvx, sbuf,
             send_sem_x, recv_sem_x, send_sem_y, recv_sem_y, copy_sem):
        k = pl.program_id(0)
        my_x = lax.axis_index("x")
        my_y = lax.axis_index("y")
        peer_x = (1 - my_x, my_y)
        peer_y = (my_x, 1 - my_y)

        def my_col(t):
            return (my_y * NT + t) * TN

        def their_col(t):
            return ((1 - my_y) * NT + t) * TN

        def x_rdma(t):
            return pltpu.make_async_remote_copy(
                src_ref=ptile.at[t % 2, pl.ds((1 - my_x) * S_HALF, S_HALF), :],
                dst_ref=recvx.at[t % 4],
                send_sem=send_sem_x.at[t % 2],
                recv_sem=recv_sem_x.at[t % 4],
                device_id=peer_x,
                device_id_type=pl.DeviceIdType.MESH,
            )

        def y_rdma(t):
            return pltpu.make_async_remote_copy(
                src_ref=sbuf.at[t % 2],
                dst_ref=out_ref.at[0, :, pl.ds(my_col(t), TN)],
                send_sem=send_sem_y.at[t % 2],
                recv_sem=recv_sem_y.at[t],
                device_id=peer_y,
                device_id_type=pl.DeviceIdType.MESH,
            )

        def y_recv(t):
            return pltpu.make_async_remote_copy(
                src_ref=sbuf.at[0],
                dst_ref=out_ref.at[0, :, pl.ds(their_col(t), TN)],
                send_sem=send_sem_y.at[0],
                recv_sem=recv_sem_y.at[t],
                device_id=peer_y,
                device_id_type=pl.DeviceIdType.MESH,
            )

        def local_store(t):
            return pltpu.make_async_copy(
                sbuf.at[t % 2],
                out_ref.at[0, :, pl.ds(my_col(t), TN)],
                copy_sem.at[t % 2],
            )

        @pl.when(k == 0)
        def _():
            barrier = pltpu.get_barrier_semaphore()
            for nbr in (peer_x, peer_y):
                pl.semaphore_signal(
                    barrier, inc=1, device_id=nbr,
                    device_id_type=pl.DeviceIdType.MESH,
                )
            pl.semaphore_wait(barrier, 2)

        @pl.when(k < NT)
        def _():
            @pl.when(k >= 2)
            def _():
                x_rdma(k - 2).wait_send()

            CS = 256
            for si in range(S // CS):
                sl = slice(si * CS, (si + 1) * CS)
                acc = jnp.dot(
                    o_ref[0, sl, 0, :], w_ref[0:D, :],
                    preferred_element_type=jnp.float32,
                )
                for h in range(1, H):
                    acc += jnp.dot(
                        o_ref[0, sl, h, :], w_ref[h * D:(h + 1) * D, :],
                        preferred_element_type=jnp.float32,
                    )
                ptile[k % 2, sl, :] = acc
            x_rdma(k).start()

        @pl.when(k >= 1)
        def _():
            t = k - 1
            x_rdma(t).wait_recv()

            @pl.when(t >= 2)
            def _():
                y_rdma(t - 2).wait_send()
                local_store(t - 2).wait()

            sbuf[t % 2, :, :] = (
                ptile[t % 2, pl.ds(my_x * S_HALF, S_HALF), :]
                + recvx[t % 4]
            )
            y_rdma(t).start()
            local_store(t).start()

        @pl.when(k == NT)
        def _():
            for t in (NT - 2, NT - 1):
                x_rdma(t).wait_send()
                y_rdma(t).wait_send()
                local_store(t).wait()
            for t in range(NT):
                y_recv(t).wait_recv()

        return None

    nt_total = N // TN

    return pl.pallas_call(
        body,
        grid=(NT + 1,),
        out_shape=jax.ShapeDtypeStruct((B, S_HALF, N), jnp.float32),
        in_specs=[
            pl.BlockSpec((1, S, H, D), lambda k: (0, 0, 0, 0)),
            pl.BlockSpec(
                (K, TN),
                lambda k: (
                    0,
                    lax.axis_index("y") * NT + jnp.minimum(k, NT - 1),
                ),
            ),
        ],
        out_specs=pl.BlockSpec(memory_space=pl.ANY),
        scratch_shapes=[
            pltpu.VMEM((2, S, TN), jnp.float32),
            pltpu.VMEM((4, S_HALF, TN), jnp.float32),
            pltpu.VMEM((2, S_HALF, TN), jnp.float32),
            pltpu.SemaphoreType.DMA((2,)),
            pltpu.SemaphoreType.DMA((4,)),
            pltpu.SemaphoreType.DMA((2,)),
            pltpu.SemaphoreType.DMA((16,)),
            pltpu.SemaphoreType.DMA((2,)),
        ],
        compiler_params=pltpu.CompilerParams(
            dimension_semantics=("arbitrary",),
            collective_id=0,
            vmem_limit_bytes=62 * 1024 * 1024,
        ),
    )(O, Wo)


# device time: 238563 ns/iter; 2.5654x vs baseline; 2.5654x over previous
import jax
import jax.numpy as jnp
from jax import lax
from jax.experimental import pallas as pl
from jax.experimental.pallas import tpu as pltpu

TN = 256
S_HALF = 1024


def kernel(O, Wo):
    B, S, H, D = O.shape
    K = H * D
    N = Wo.shape[1]
    NT = (N // 2) // TN

    def body(o_ref, w_ref, out_ref, a_buf, ptile, recvx, sbuf,
             send_sem_x, recv_sem_x, send_sem_y, recv_sem_y, copy_sem,
             stage_sem):
        k = pl.program_id(0)
        my_x = lax.axis_index("x")
        my_y = lax.axis_index("y")
        peer_x = (1 - my_x, my_y)
        peer_y = (my_x, 1 - my_y)

        def my_col(t):
            return (my_y * NT + t) * TN

        def their_col(t):
            return ((1 - my_y) * NT + t) * TN

        def x_rdma(t):
            return pltpu.make_async_remote_copy(
                src_ref=ptile.at[t % 2, pl.ds((1 - my_x) * S_HALF, S_HALF), :],
                dst_ref=recvx.at[t % 4],
                send_sem=send_sem_x.at[t % 2],
                recv_sem=recv_sem_x.at[t % 4],
                device_id=peer_x,
                device_id_type=pl.DeviceIdType.MESH,
            )

        def y_rdma(t):
            return pltpu.make_async_remote_copy(
                src_ref=sbuf.at[t % 2],
                dst_ref=out_ref.at[0, :, pl.ds(my_col(t), TN)],
                send_sem=send_sem_y.at[t % 2],
                recv_sem=recv_sem_y.at[t],
                device_id=peer_y,
                device_id_type=pl.DeviceIdType.MESH,
            )

        def y_recv(t):
            return pltpu.make_async_remote_copy(
                src_ref=sbuf.at[0],
                dst_ref=out_ref.at[0, :, pl.ds(their_col(t), TN)],
                send_sem=send_sem_y.at[0],
                recv_sem=recv_sem_y.at[t],
                device_id=peer_y,
                device_id_type=pl.DeviceIdType.MESH,
            )

        def local_store(t):
            return pltpu.make_async_copy(
                sbuf.at[t % 2],
                out_ref.at[0, :, pl.ds(my_col(t), TN)],
                copy_sem.at[t % 2],
            )

        def stage_copy(h):
            return pltpu.make_async_copy(
                o_ref.at[0, :, h, :],
                a_buf.at[:, pl.ds(h * D, D)],
                stage_sem.at[h],
            )

        @pl.when(k == 0)
        def _():
            for h in range(H):
                stage_copy(h).start()
            barrier = pltpu.get_barrier_semaphore()
            for nbr in (peer_x, peer_y):
                pl.semaphore_signal(
                    barrier, inc=1, device_id=nbr,
                    device_id_type=pl.DeviceIdType.MESH,
                )
            pl.semaphore_wait(barrier, 2)
            for h in range(H):
                stage_copy(h).wait()

        @pl.when(k < NT)
        def _():
            @pl.when(k >= 2)
            def _():
                x_rdma(k - 2).wait_send()

            ptile[k % 2, :, :] = jnp.dot(
                a_buf[...], w_ref[...], preferred_element_type=jnp.float32
            )
            x_rdma(k).start()

        @pl.when(k >= 1)
        def _():
            t = k - 1
            x_rdma(t).wait_recv()

            @pl.when(t >= 2)
            def _():
                y_rdma(t - 2).wait_send()
                local_store(t - 2).wait()

            sbuf[t % 2, :, :] = (
                ptile[t % 2, pl.ds(my_x * S_HALF, S_HALF), :]
                + recvx[t % 4]
            )
            y_rdma(t).start()
            local_store(t).start()

        @pl.when(k == NT)
        def _():
            for t in (NT - 2, NT - 1):
                x_rdma(t).wait_send()
                y_rdma(t).wait_send()
                local_store(t).wait()
            for t in range(NT):
                y_recv(t).wait_recv()

        return None

    nt_total = N // TN

    return pl.pallas_call(
        body,
        grid=(NT + 1,),
        out_shape=jax.ShapeDtypeStruct((B, S_HALF, N), jnp.float32),
        in_specs=[
            pl.BlockSpec(memory_space=pl.ANY),
            pl.BlockSpec(
                (K, TN),
                lambda k: (
                    0,
                    lax.axis_index("y") * NT + jnp.minimum(k, NT - 1),
                ),
            ),
        ],
        out_specs=pl.BlockSpec(memory_space=pl.ANY),
        scratch_shapes=[
            pltpu.VMEM((S, K), jnp.float32),
            pltpu.VMEM((2, S, TN), jnp.float32),
            pltpu.VMEM((4, S_HALF, TN), jnp.float32),
            pltpu.VMEM((2, S_HALF, TN), jnp.float32),
            pltpu.SemaphoreType.DMA((2,)),
            pltpu.SemaphoreType.DMA((4,)),
            pltpu.SemaphoreType.DMA((2,)),
            pltpu.SemaphoreType.DMA((16,)),
            pltpu.SemaphoreType.DMA((2,)),
            pltpu.SemaphoreType.DMA((H,)),
        ],
        compiler_params=pltpu.CompilerParams(
            dimension_semantics=("arbitrary",),
            collective_id=0,
            vmem_limit_bytes=62 * 1024 * 1024,
        ),
    )(O, Wo)


# device time: 156215 ns/iter; 3.9177x vs baseline; 1.5271x over previous
import jax
import jax.numpy as jnp
from jax import lax
from jax.experimental import pallas as pl
from jax.experimental.pallas import tpu as pltpu

TN = 256
S_HALF = 1024


def kernel(O, Wo):
    B, S, H, D = O.shape
    K = H * D
    N = Wo.shape[1]
    NT = (N // 2) // TN

    def body(o_ref, w_ref, out_ref, a_buf, ptile, recvx, sbuf, ybuf,
             recvy, obuf, send_sem_x, recv_sem_x, send_sem_y, recv_sem_y,
             copy_sem, copy2_sem, stage_sem):
        k = pl.program_id(0)
        my_x = lax.axis_index("x")
        my_y = lax.axis_index("y")
        peer_x = (1 - my_x, my_y)
        peer_y = (my_x, 1 - my_y)

        def my_col(t):
            return (my_y * NT + t) * TN

        def their_col(t):
            return ((1 - my_y) * NT + t) * TN

        def x_rdma(t):
            return pltpu.make_async_remote_copy(
                src_ref=ptile.at[t % 2, pl.ds((1 - my_x) * S_HALF, S_HALF), :],
                dst_ref=recvx.at[t % 4],
                send_sem=send_sem_x.at[t % 2],
                recv_sem=recv_sem_x.at[t % 4],
                device_id=peer_x,
                device_id_type=pl.DeviceIdType.MESH,
            )

        def y_rdma(t):
            return pltpu.make_async_remote_copy(
                src_ref=ybuf.at[t % 2],
                dst_ref=recvy.at[t % 6],
                send_sem=send_sem_y.at[t % 2],
                recv_sem=recv_sem_y.at[t % 6],
                device_id=peer_y,
                device_id_type=pl.DeviceIdType.MESH,
            )

        def local_store(t):
            return pltpu.make_async_copy(
                sbuf.at[t % 2],
                out_ref.at[0, :, pl.ds(my_col(t), TN)],
                copy_sem.at[t % 2],
            )

        def peer_store(t):
            return pltpu.make_async_copy(
                obuf.at[t % 2],
                out_ref.at[0, :, pl.ds(their_col(t), TN)],
                copy2_sem.at[t % 2],
            )

        def stage_copy(h):
            return pltpu.make_async_copy(
                o_ref.at[0, :, h, :],
                a_buf.at[:, pl.ds(h * D, D)],
                stage_sem.at[h],
            )

        @pl.when(k == 0)
        def _():
            for h in range(H):
                stage_copy(h).start()
            barrier = pltpu.get_barrier_semaphore()
            for nbr in (peer_x, peer_y):
                pl.semaphore_signal(
                    barrier, inc=1, device_id=nbr,
                    device_id_type=pl.DeviceIdType.MESH,
                )
            pl.semaphore_wait(barrier, 2)
            for h in range(H):
                stage_copy(h).wait()

        @pl.when(k < NT)
        def _():
            @pl.when(k >= 2)
            def _():
                x_rdma(k - 2).wait_send()

            ptile[k % 2, :, :] = jnp.dot(
                a_buf[...], w_ref[...], preferred_element_type=jnp.float32
            ).astype(jnp.bfloat16)
            x_rdma(k).start()

        @pl.when(jnp.logical_and(k >= 1, k <= NT))
        def _():
            t = k - 1
            x_rdma(t).wait_recv()

            @pl.when(t >= 2)
            def _():
                y_rdma(t - 2).wait_send()
                local_store(t - 2).wait()

            ssum = (
                ptile[t % 2, pl.ds(my_x * S_HALF, S_HALF), :].astype(
                    jnp.float32
                )
                + recvx[t % 4].astype(jnp.float32)
            )
            sbuf[t % 2, :, :] = ssum
            ybuf[t % 2, :, :] = ssum.astype(jnp.bfloat16)
            y_rdma(t).start()
            local_store(t).start()

        @pl.when(k >= 2)
        def _():
            t2 = k - 2
            y_rdma(t2).wait_recv()

            @pl.when(t2 >= 2)
            def _():
                peer_store(t2 - 2).wait()

            obuf[t2 % 2, :, :] = recvy[t2 % 6].astype(jnp.float32)
            peer_store(t2).start()

        @pl.when(k == NT + 1)
        def _():
            for t in (NT - 2, NT - 1):
                x_rdma(t).wait_send()
                y_rdma(t).wait_send()
                local_store(t).wait()
                peer_store(t).wait()

        return None

    return pl.pallas_call(
        body,
        grid=(NT + 2,),
        out_shape=jax.ShapeDtypeStruct((B, S_HALF, N), jnp.float32),
        in_specs=[
            pl.BlockSpec(memory_space=pl.ANY),
            pl.BlockSpec(
                (K, TN),
                lambda k: (
                    0,
                    lax.axis_index("y") * NT + jnp.minimum(k, NT - 1),
                ),
            ),
        ],
        out_specs=pl.BlockSpec(memory_space=pl.ANY),
        scratch_shapes=[
            pltpu.VMEM((S, K), jnp.float32),
            pltpu.VMEM((2, S, TN), jnp.bfloat16),
            pltpu.VMEM((4, S_HALF, TN), jnp.bfloat16),
            pltpu.VMEM((2, S_HALF, TN), jnp.float32),
            pltpu.VMEM((2, S_HALF, TN), jnp.bfloat16),
            pltpu.VMEM((6, S_HALF, TN), jnp.bfloat16),
            pltpu.VMEM((2, S_HALF, TN), jnp.float32),
            pltpu.SemaphoreType.DMA((2,)),
            pltpu.SemaphoreType.DMA((4,)),
            pltpu.SemaphoreType.DMA((2,)),
            pltpu.SemaphoreType.DMA((6,)),
            pltpu.SemaphoreType.DMA((2,)),
            pltpu.SemaphoreType.DMA((2,)),
            pltpu.SemaphoreType.DMA((H,)),
        ],
        compiler_params=pltpu.CompilerParams(
            dimension_semantics=("arbitrary",),
            collective_id=0,
            vmem_limit_bytes=63 * 1024 * 1024,
        ),
    )(O, Wo)
